# baseline (speedup 1.0000x reference)
import dataclasses
import functools

import jax
import jax.numpy as jnp
from jax import lax
from jax.experimental import pallas as pl
from jax.experimental.pallas import tpu as pltpu
from jax.experimental.pallas import tpu_sc as plsc

N = 10000
D = 128
E = 320000
NP = 10112
NTILES = 32
C = 96
CH0 = 124
CH1 = 88
EP = 16 * (CH0 + CH1) * C
QB = 4
MAXBLK = CH0 // QB
RPS = NP // 16
EDT = E // NTILES

_MESH = plsc.VectorSubcoreMesh(core_axis_name="c", subcore_axis_name="s")

_CP = pltpu.CompilerParams()
if "needs_layout_passes" in pltpu.CompilerParams.__dataclass_fields__:
  _CP = dataclasses.replace(_CP, needs_layout_passes=False)

_RPS_PIECES = []
_r = 0
while _r < RPS:
  _RPS_PIECES.append((min(C, RPS - _r), _r))
  _r += min(C, RPS - _r)


@functools.partial(
    pl.kernel,
    out_type=jax.ShapeDtypeStruct((2, NP, D), jnp.float32),
    mesh=_MESH,
    scratch_types=[
        pltpu.VMEM((QB, 2, C), jnp.int32),
        pltpu.VMEM((C, D), jnp.float32),
        pltpu.VMEM((C, D), jnp.float32),
        pltpu.VMEM_SHARED((NP, D), jnp.float32),
        pltpu.SemaphoreType.DMA,
        pltpu.SemaphoreType.DMA,
        pltpu.SemaphoreType.DMA,
        pltpu.SemaphoreType.DMA,
    ])
def _segsum(x_hbm, idx_hbm, out_hbm,
            idx4, buf0, buf1, acc, g0, g1, sa, sb):
  cid = lax.axis_index("c")
  sid = lax.axis_index("s")
  wid = cid * 16 + sid

  @pl.loop(0, C)
  def _(i):
    for c in range(D // 16):
      buf0.at[pl.ds(i, 1), pl.ds(c * 16, 16)][...] = jnp.zeros(
          (1, 16), jnp.float32)

  for sz, off in _RPS_PIECES:
    pltpu.sync_copy(buf0.at[pl.ds(0, sz)], acc.at[pl.ds(sid * RPS + off, sz)])
  plsc.subcore_barrier()

  cbase = jnp.where(cid == 0, sid * CH0, 16 * CH0 + sid * CH1)
  myblocks = jnp.where(cid == 0, CH0 // QB, CH1 // QB)

  @pl.loop(0, MAXBLK)
  def _(jb):
    @pl.when(jb < myblocks)
    def _():
      c0 = cbase + jb * QB
      pltpu.sync_copy(idx_hbm.at[pl.ds(c0, QB)], idx4)
      i0, i1, i2, i3 = idx4.at[0], idx4.at[1], idx4.at[2], idx4.at[3]
      h0 = pltpu.async_copy(x_hbm.at[i0.at[1]], buf0, g0)
      h1 = pltpu.async_copy(x_hbm.at[i1.at[1]], buf1, g1)
      h0.wait()
      hs0 = pltpu.async_copy(buf0, acc.at[i0.at[0]], sa, add=True)
      h1.wait()
      hs1 = pltpu.async_copy(buf1, acc.at[i1.at[0]], sb, add=True)
      hs0.wait()
      h0 = pltpu.async_copy(x_hbm.at[i2.at[1]], buf0, g0)
      hs1.wait()
      h1 = pltpu.async_copy(x_hbm.at[i3.at[1]], buf1, g1)
      h0.wait()
      hs0 = pltpu.async_copy(buf0, acc.at[i2.at[0]], sa, add=True)
      h1.wait()
      hs1 = pltpu.async_copy(buf1, acc.at[i3.at[0]], sb, add=True)
      hs0.wait()
      hs1.wait()

  plsc.subcore_barrier()
  for sz, off in _RPS_PIECES:
    r0 = sid * RPS + off
    pltpu.sync_copy(acc.at[pl.ds(r0, sz)], out_hbm.at[cid].at[pl.ds(r0, sz)])


@functools.partial(
    pl.kernel,
    out_type=jax.ShapeDtypeStruct((NTILES, NP), jnp.float32),
    mesh=_MESH,
    compiler_params=_CP,
    scratch_types=[
        pltpu.VMEM((EDT,), jnp.int32),
        pltpu.VMEM((NP,), jnp.float32),
    ])
def _deghist(dst_hbm, out_hbm, didx, hist):
  cid = lax.axis_index("c")
  sid = lax.axis_index("s")
  wid = cid * 16 + sid

  @pl.loop(0, NP // 16)
  def _(i):
    hist.at[pl.ds(i * 16, 16)][...] = jnp.zeros((16,), jnp.float32)

  pltpu.sync_copy(dst_hbm.at[pl.ds(wid * EDT, EDT)], didx)
  ones16 = jnp.ones((16,), jnp.float32)

  @pl.loop(0, EDT // 16)
  def _(k):
    idx = didx[pl.ds(k * 16, 16)]
    plsc.addupdate_scatter(hist, [idx], ones16)

  pltpu.sync_copy(hist, out_hbm.at[wid])


def _make_tc(with_relu: bool, final: bool):
  def body(p_ref, dg_ref, w_ref, b_ref, *rest):
    if final:
      w3_ref, b3_ref, o_ref = rest
    else:
      (o_ref,) = rest
    g = p_ref[0] + p_ref[1]
    deg = jnp.sum(dg_ref[...], axis=0)[:, None]
    xx = jnp.dot(g, w_ref[...], preferred_element_type=jnp.float32,
                 precision=lax.Precision.HIGHEST)
    xx = xx + deg * b_ref[...]
    if with_relu:
      xx = jnp.maximum(xx, 0.0)
    if final:
      xx = jnp.dot(xx, w3_ref[...], preferred_element_type=jnp.float32,
                   precision=lax.Precision.HIGHEST) + b3_ref[...]
    o_ref[...] = xx

  return pl.pallas_call(
      body, out_shape=jax.ShapeDtypeStruct((NP, D), jnp.float32))


_tc_step = _make_tc(False, False)
_tc_step_relu = _make_tc(True, False)
_tc_final = _make_tc(True, True)


def kernel(x, edge_index, W1, b1, W2, b2, W3, b3):
  src = edge_index[0].astype(jnp.int32)
  dst = edge_index[1].astype(jnp.int32)
  pad = EP - E
  pad_src = jnp.full((pad,), N, jnp.int32)
  pad_dst = jnp.arange(pad, dtype=jnp.int32) % N
  src1 = jnp.concatenate([src, pad_src]).reshape(EP // C, C)
  dst1 = jnp.concatenate([dst, pad_dst]).reshape(EP // C, C)
  idx2 = jnp.stack([dst1, src1], axis=1)
  xp = jnp.pad(x, ((0, NP - N), (0, 0)))
  b1r, b2r, b3r = b1.reshape(1, D), b2.reshape(1, D), b3.reshape(1, D)

  degp = _deghist(dst)
  P = _segsum(xp, idx2)
  x1 = _tc_step(P, degp, W1, b1r)
  P = _segsum(x1, idx2)
  x2 = _tc_step_relu(P, degp, W1, b1r)
  P = _segsum(x2, idx2)
  x3 = _tc_step(P, degp, W2, b2r)
  P = _segsum(x3, idx2)
  out = _tc_final(P, degp, W2, b2r, W3, b3r)
  return out[:N]


# speedup vs baseline: 1.4969x; 1.4969x over previous
import dataclasses
import functools

import jax
import jax.numpy as jnp
from jax import lax
from jax.experimental import pallas as pl
from jax.experimental.pallas import tpu as pltpu
from jax.experimental.pallas import tpu_sc as plsc

N = 10000
D = 128
E = 320000
NP = 10112
NTILES = 32
C = 96
CH0 = 122
CH1 = 88
EP = 16 * (CH0 + CH1) * C
MAXPAIRS = CH0 // 2
RPS = NP // 16
EDT = E // NTILES

_MESH = plsc.VectorSubcoreMesh(core_axis_name="c", subcore_axis_name="s")

_CP = pltpu.CompilerParams()
if "needs_layout_passes" in pltpu.CompilerParams.__dataclass_fields__:
  _CP = dataclasses.replace(_CP, needs_layout_passes=False)

_RPS_PIECES = []
_r = 0
while _r < RPS:
  _RPS_PIECES.append((min(C, RPS - _r), _r))
  _r += min(C, RPS - _r)


@functools.partial(
    pl.kernel,
    out_type=jax.ShapeDtypeStruct((2, NP, D), jnp.float32),
    mesh=_MESH,
    scratch_types=[
        pltpu.VMEM((2, C), jnp.int32),
        pltpu.VMEM((2, C), jnp.int32),
        pltpu.VMEM((C, D), jnp.float32),
        pltpu.VMEM((C, D), jnp.float32),
        pltpu.VMEM_SHARED((NP, D), jnp.float32),
        pltpu.SemaphoreType.DMA,
        pltpu.SemaphoreType.DMA,
        pltpu.SemaphoreType.DMA,
        pltpu.SemaphoreType.DMA,
    ])
def _segsum(x_hbm, idx_hbm, out_hbm,
            i0, i1, buf0, buf1, acc, g0, g1, sa, sb):
  cid = lax.axis_index("c")
  sid = lax.axis_index("s")
  wid = cid * 16 + sid

  @pl.loop(0, C)
  def _(i):
    for c in range(D // 16):
      buf0.at[pl.ds(i, 1), pl.ds(c * 16, 16)][...] = jnp.zeros(
          (1, 16), jnp.float32)

  for sz, off in _RPS_PIECES:
    pltpu.sync_copy(buf0.at[pl.ds(0, sz)], acc.at[pl.ds(sid * RPS + off, sz)])
  plsc.subcore_barrier()

  cbase = jnp.where(cid == 0, sid * CH0, 16 * CH0 + sid * CH1)
  mypairs = jnp.where(cid == 0, CH0 // 2, CH1 // 2)

  @pl.loop(0, MAXPAIRS)
  def _(j2):
    @pl.when(j2 < mypairs)
    def _():
      c0 = cbase + j2 * 2
      pltpu.sync_copy(idx_hbm.at[c0], i0)
      h0 = pltpu.async_copy(x_hbm.at[i0.at[1]], buf0, g0)
      pltpu.sync_copy(idx_hbm.at[c0 + 1], i1)
      h1 = pltpu.async_copy(x_hbm.at[i1.at[1]], buf1, g1)
      h0.wait()
      hs0 = pltpu.async_copy(buf0, acc.at[i0.at[0]], sa, add=True)
      h1.wait()
      hs1 = pltpu.async_copy(buf1, acc.at[i1.at[0]], sb, add=True)
      hs0.wait()
      hs1.wait()

  plsc.subcore_barrier()
  for sz, off in _RPS_PIECES:
    r0 = sid * RPS + off
    pltpu.sync_copy(acc.at[pl.ds(r0, sz)], out_hbm.at[cid].at[pl.ds(r0, sz)])


@functools.partial(
    pl.kernel,
    out_type=jax.ShapeDtypeStruct((NTILES, NP), jnp.float32),
    mesh=_MESH,
    compiler_params=_CP,
    scratch_types=[
        pltpu.VMEM((EDT,), jnp.int32),
        pltpu.VMEM((NP,), jnp.float32),
    ])
def _deghist(dst_hbm, out_hbm, didx, hist):
  cid = lax.axis_index("c")
  sid = lax.axis_index("s")
  wid = cid * 16 + sid

  @pl.loop(0, NP // 16)
  def _(i):
    hist.at[pl.ds(i * 16, 16)][...] = jnp.zeros((16,), jnp.float32)

  pltpu.sync_copy(dst_hbm.at[pl.ds(wid * EDT, EDT)], didx)
  ones16 = jnp.ones((16,), jnp.float32)

  @pl.loop(0, EDT // 16)
  def _(k):
    idx = didx[pl.ds(k * 16, 16)]
    plsc.addupdate_scatter(hist, [idx], ones16)

  pltpu.sync_copy(hist, out_hbm.at[wid])


def _make_tc(with_relu: bool, final: bool):
  def body(p_ref, dg_ref, w_ref, b_ref, *rest):
    if final:
      w3_ref, b3_ref, o_ref = rest
    else:
      (o_ref,) = rest
    g = p_ref[0] + p_ref[1]
    deg = jnp.sum(dg_ref[...], axis=0)[:, None]
    xx = jnp.dot(g, w_ref[...], preferred_element_type=jnp.float32,
                 precision=lax.Precision.HIGHEST)
    xx = xx + deg * b_ref[...]
    if with_relu:
      xx = jnp.maximum(xx, 0.0)
    if final:
      xx = jnp.dot(xx, w3_ref[...], preferred_element_type=jnp.float32,
                   precision=lax.Precision.HIGHEST) + b3_ref[...]
    o_ref[...] = xx

  return pl.pallas_call(
      body, out_shape=jax.ShapeDtypeStruct((NP, D), jnp.float32))


_tc_step = _make_tc(False, False)
_tc_step_relu = _make_tc(True, False)
_tc_final = _make_tc(True, True)


def kernel(x, edge_index, W1, b1, W2, b2, W3, b3):
  src = edge_index[0].astype(jnp.int32)
  dst = edge_index[1].astype(jnp.int32)
  pad = EP - E
  pad_src = jnp.full((pad,), N, jnp.int32)
  pad_dst = jnp.arange(pad, dtype=jnp.int32) % N
  src1 = jnp.concatenate([src, pad_src]).reshape(EP // C, C)
  dst1 = jnp.concatenate([dst, pad_dst]).reshape(EP // C, C)
  idx2 = jnp.stack([dst1, src1], axis=1)
  xp = jnp.pad(x, ((0, NP - N), (0, 0)))
  b1r, b2r, b3r = b1.reshape(1, D), b2.reshape(1, D), b3.reshape(1, D)

  degp = _deghist(dst)
  P = _segsum(xp, idx2)
  x1 = _tc_step(P, degp, W1, b1r)
  P = _segsum(x1, idx2)
  x2 = _tc_step_relu(P, degp, W1, b1r)
  P = _segsum(x2, idx2)
  x3 = _tc_step(P, degp, W2, b2r)
  P = _segsum(x3, idx2)
  out = _tc_final(P, degp, W2, b2r, W3, b3r)
  return out[:N]


# speedup vs baseline: 1.5272x; 1.0203x over previous
import dataclasses
import functools

import jax
import jax.numpy as jnp
from jax import lax
from jax.experimental import pallas as pl
from jax.experimental.pallas import tpu as pltpu
from jax.experimental.pallas import tpu_sc as plsc

N = 10000
D = 128
E = 320000
NP = 10112
NTILES = 32
C = 96
CH0 = 128
CH1 = 82
EP = 16 * (CH0 + CH1) * C
MAXPAIRS = CH0 // 2
RPS = NP // 16
EDT = E // NTILES

_MESH = plsc.VectorSubcoreMesh(core_axis_name="c", subcore_axis_name="s")

_CP = pltpu.CompilerParams()
if "needs_layout_passes" in pltpu.CompilerParams.__dataclass_fields__:
  _CP = dataclasses.replace(_CP, needs_layout_passes=False)

_RPS_PIECES = []
_r = 0
while _r < RPS:
  _RPS_PIECES.append((min(C, RPS - _r), _r))
  _r += min(C, RPS - _r)


@functools.partial(
    pl.kernel,
    out_type=jax.ShapeDtypeStruct((2, NP, D), jnp.float32),
    mesh=_MESH,
    scratch_types=[
        pltpu.VMEM((2, C), jnp.int32),
        pltpu.VMEM((2, C), jnp.int32),
        pltpu.VMEM((C, D), jnp.float32),
        pltpu.VMEM((C, D), jnp.float32),
        pltpu.VMEM_SHARED((NP, D), jnp.float32),
        pltpu.SemaphoreType.DMA,
        pltpu.SemaphoreType.DMA,
        pltpu.SemaphoreType.DMA,
        pltpu.SemaphoreType.DMA,
    ])
def _segsum(x_hbm, idx_hbm, out_hbm,
            i0, i1, buf0, buf1, acc, g0, g1, sa, sb):
  cid = lax.axis_index("c")
  sid = lax.axis_index("s")
  wid = cid * 16 + sid

  @pl.loop(0, C)
  def _(i):
    for c in range(D // 16):
      buf0.at[pl.ds(i, 1), pl.ds(c * 16, 16)][...] = jnp.zeros(
          (1, 16), jnp.float32)

  for sz, off in _RPS_PIECES:
    pltpu.sync_copy(buf0.at[pl.ds(0, sz)], acc.at[pl.ds(sid * RPS + off, sz)])
  plsc.subcore_barrier()

  cbase = jnp.where(cid == 0, sid * CH0, 16 * CH0 + sid * CH1)
  mypairs = jnp.where(cid == 0, CH0 // 2, CH1 // 2)

  @pl.loop(0, MAXPAIRS)
  def _(j2):
    @pl.when(j2 < mypairs)
    def _():
      c0 = cbase + j2 * 2
      pltpu.sync_copy(idx_hbm.at[c0], i0)
      h0 = pltpu.async_copy(x_hbm.at[i0.at[1]], buf0, g0)
      pltpu.sync_copy(idx_hbm.at[c0 + 1], i1)
      h1 = pltpu.async_copy(x_hbm.at[i1.at[1]], buf1, g1)
      h0.wait()
      hs0 = pltpu.async_copy(buf0, acc.at[i0.at[0]], sa, add=True)
      h1.wait()
      hs1 = pltpu.async_copy(buf1, acc.at[i1.at[0]], sb, add=True)
      hs0.wait()
      hs1.wait()

  plsc.subcore_barrier()
  for sz, off in _RPS_PIECES:
    r0 = sid * RPS + off
    pltpu.sync_copy(acc.at[pl.ds(r0, sz)], out_hbm.at[cid].at[pl.ds(r0, sz)])


@functools.partial(
    pl.kernel,
    out_type=jax.ShapeDtypeStruct((NTILES, NP), jnp.float32),
    mesh=_MESH,
    compiler_params=_CP,
    scratch_types=[
        pltpu.VMEM((EDT,), jnp.int32),
        pltpu.VMEM((NP,), jnp.float32),
    ])
def _deghist(dst_hbm, out_hbm, didx, hist):
  cid = lax.axis_index("c")
  sid = lax.axis_index("s")
  wid = cid * 16 + sid

  @pl.loop(0, NP // 16)
  def _(i):
    hist.at[pl.ds(i * 16, 16)][...] = jnp.zeros((16,), jnp.float32)

  pltpu.sync_copy(dst_hbm.at[pl.ds(wid * EDT, EDT)], didx)
  ones16 = jnp.ones((16,), jnp.float32)

  @pl.loop(0, EDT // 16)
  def _(k):
    idx = didx[pl.ds(k * 16, 16)]
    plsc.addupdate_scatter(hist, [idx], ones16)

  pltpu.sync_copy(hist, out_hbm.at[wid])


def _make_tc(with_relu: bool, final: bool):
  def body(p_ref, dg_ref, w_ref, b_ref, *rest):
    if final:
      w3_ref, b3_ref, o_ref = rest
    else:
      (o_ref,) = rest
    g = p_ref[0] + p_ref[1]
    deg = jnp.sum(dg_ref[...], axis=0)[:, None]
    xx = jnp.dot(g, w_ref[...], preferred_element_type=jnp.float32,
                 precision=lax.Precision.HIGHEST)
    xx = xx + deg * b_ref[...]
    if with_relu:
      xx = jnp.maximum(xx, 0.0)
    if final:
      xx = jnp.dot(xx, w3_ref[...], preferred_element_type=jnp.float32,
                   precision=lax.Precision.HIGHEST) + b3_ref[...]
    o_ref[...] = xx

  return pl.pallas_call(
      body, out_shape=jax.ShapeDtypeStruct((NP, D), jnp.float32))


_tc_step = _make_tc(False, False)
_tc_step_relu = _make_tc(True, False)
_tc_final = _make_tc(True, True)


def kernel(x, edge_index, W1, b1, W2, b2, W3, b3):
  src = edge_index[0].astype(jnp.int32)
  dst = edge_index[1].astype(jnp.int32)
  pad = EP - E
  pad_src = jnp.full((pad,), N, jnp.int32)
  pad_dst = jnp.arange(pad, dtype=jnp.int32) % N
  src1 = jnp.concatenate([src, pad_src]).reshape(EP // C, C)
  dst1 = jnp.concatenate([dst, pad_dst]).reshape(EP // C, C)
  idx2 = jnp.stack([dst1, src1], axis=1)
  xp = jnp.pad(x, ((0, NP - N), (0, 0)))
  b1r, b2r, b3r = b1.reshape(1, D), b2.reshape(1, D), b3.reshape(1, D)

  degp = _deghist(dst)
  P = _segsum(xp, idx2)
  x1 = _tc_step(P, degp, W1, b1r)
  P = _segsum(x1, idx2)
  x2 = _tc_step_relu(P, degp, W1, b1r)
  P = _segsum(x2, idx2)
  x3 = _tc_step(P, degp, W2, b2r)
  P = _segsum(x3, idx2)
  out = _tc_final(P, degp, W2, b2r, W3, b3r)
  return out[:N]
